# Initial kernel scaffold; baseline (speedup 1.0000x reference)
import jax
import jax.numpy as jnp
from jax.experimental import pallas as pl


def kernel(x, edge_index, W1, b1, W2, b2):
    raise NotImplementedError("write your pallas kernel here")


# speedup vs baseline: 23.2206x; 23.2206x over previous
import functools

import jax
import jax.numpy as jnp
from jax import lax
from jax.experimental import pallas as pl
from jax.experimental.pallas import tpu as pltpu
from jax.experimental.pallas import tpu_sc as plsc

N = 10000
NPAD = 10240
D = 128
E = 320000
NC, NS = 2, 16
NW = NC * NS
EC = 128
SUP = 8
EPT = -(-E // NW)
NSUP = -(-EPT // (SUP * EC))
EPT_PAD = NSUP * SUP * EC
ROWS_PT = NPAD // NS
RB = 1280
GRID = NPAD // RB

_mesh = plsc.VectorSubcoreMesh(core_axis_name="c", subcore_axis_name="s")


def _deg_body(dst_hbm, deg_out, dst_v, ones_v, zero_v, deg_sh):
    c = lax.axis_index("c")
    s = lax.axis_index("s")
    wid = s * NC + c
    row0 = s * ROWS_PT

    one = jnp.full((16,), 1.0, jnp.float32)
    zero = jnp.zeros((16,), jnp.float32)
    for i in range(EC):
        for j in range(D // 16):
            ones_v[i, pl.ds(j * 16, 16)] = one
    for i in range(16):
        for j in range(D // 16):
            zero_v[i, pl.ds(j * 16, 16)] = zero

    for k in range(ROWS_PT // 16):
        pltpu.sync_copy(zero_v, deg_sh.at[pl.ds(row0 + k * 16, 16)])
    plsc.subcore_barrier()

    def outer(u, carry):
        pltpu.sync_copy(dst_hbm.at[wid * NSUP + u], dst_v)
        for j in range(SUP):
            pltpu.sync_copy(ones_v, deg_sh.at[dst_v.at[j]], add=True)
        return carry

    lax.fori_loop(0, NSUP, outer, 0)
    plsc.subcore_barrier()

    pltpu.sync_copy(deg_sh.at[pl.ds(row0, ROWS_PT)],
                    deg_out.at[pl.ds(c * NPAD + row0, ROWS_PT)])


_deg_call = functools.partial(
    pl.kernel,
    out_type=jax.ShapeDtypeStruct((NC * NPAD, D), jnp.float32),
    mesh=_mesh,
    scratch_types=[
        pltpu.VMEM((SUP, EC), jnp.int32),
        pltpu.VMEM((EC, D), jnp.float32),
        pltpu.VMEM((16, D), jnp.float32),
        pltpu.VMEM_SHARED((NPAD, D), jnp.float32),
    ],
)(_deg_body)


def _scat_body(g_hbm, src_hbm, dst_hbm, acc_out, src_v, dst_v, rows_a, rows_b,
               zero_v, acc_sh, sem_a, sem_b):
    c = lax.axis_index("c")
    s = lax.axis_index("s")
    wid = s * NC + c
    row0 = s * ROWS_PT

    zero = jnp.zeros((16,), jnp.float32)
    for i in range(16):
        for j in range(D // 16):
            zero_v[i, pl.ds(j * 16, 16)] = zero
    for k in range(ROWS_PT // 16):
        pltpu.sync_copy(zero_v, acc_sh.at[pl.ds(row0 + k * 16, 16)])
    plsc.subcore_barrier()

    def outer(u, carry):
        pltpu.sync_copy(src_hbm.at[wid * NSUP + u], src_v)
        pltpu.sync_copy(dst_hbm.at[wid * NSUP + u], dst_v)
        pltpu.async_copy(g_hbm.at[src_v.at[0]], rows_a, sem_a)
        for j in range(SUP):
            cur, csem = (rows_a, sem_a) if j % 2 == 0 else (rows_b, sem_b)
            nxt, xsem = (rows_b, sem_b) if j % 2 == 0 else (rows_a, sem_a)
            if j + 1 < SUP:
                pltpu.async_copy(g_hbm.at[src_v.at[j + 1]], nxt, xsem)
            pltpu.make_async_copy(g_hbm.at[src_v.at[j]], cur, csem).wait()
            pltpu.sync_copy(cur, acc_sh.at[dst_v.at[j]], add=True)
        return carry

    lax.fori_loop(0, NSUP, outer, 0)
    plsc.subcore_barrier()

    pltpu.sync_copy(acc_sh.at[pl.ds(row0, ROWS_PT)],
                    acc_out.at[pl.ds(c * NPAD + row0, ROWS_PT)])


_scat_call = functools.partial(
    pl.kernel,
    out_type=jax.ShapeDtypeStruct((NC * NPAD, D), jnp.float32),
    mesh=_mesh,
    scratch_types=[
        pltpu.VMEM((SUP, EC), jnp.int32),
        pltpu.VMEM((SUP, EC), jnp.int32),
        pltpu.VMEM((EC, D), jnp.float32),
        pltpu.VMEM((EC, D), jnp.float32),
        pltpu.VMEM((16, D), jnp.float32),
        pltpu.VMEM_SHARED((NPAD, D), jnp.float32),
        pltpu.SemaphoreType.DMA,
        pltpu.SemaphoreType.DMA,
    ],
)(_scat_body)


def _tc1_body(x_ref, w_ref, d0_ref, d1_ref, g_ref, dinv_ref):
    deg = jnp.maximum(d0_ref[:, 0:1] + d1_ref[:, 0:1] + 1.0, 1.0)
    dinv = lax.rsqrt(deg)
    h = jnp.dot(x_ref[...], w_ref[...], preferred_element_type=jnp.float32)
    g_ref[...] = h * dinv
    dinv_ref[...] = jnp.broadcast_to(dinv, dinv_ref.shape)


def _tc1(x, w1, d0, d1):
    return pl.pallas_call(
        _tc1_body,
        grid=(GRID,),
        in_specs=[
            pl.BlockSpec((RB, D), lambda i: (i, 0)),
            pl.BlockSpec((D, D), lambda i: (0, 0)),
            pl.BlockSpec((RB, D), lambda i: (i, 0)),
            pl.BlockSpec((RB, D), lambda i: (i, 0)),
        ],
        out_specs=[
            pl.BlockSpec((RB, D), lambda i: (i, 0)),
            pl.BlockSpec((RB, 16), lambda i: (i, 0)),
        ],
        out_shape=[
            jax.ShapeDtypeStruct((NPAD, D), jnp.float32),
            jax.ShapeDtypeStruct((NPAD, 16), jnp.float32),
        ],
    )(x, w1, d0, d1)


def _tc2_body(a0_ref, a1_ref, g_ref, dinv_ref, b_ref, w_ref, g2_ref):
    i = pl.program_id(0)
    dinv = dinv_ref[:, 0:1]
    z = dinv * (a0_ref[...] + a1_ref[...] + g_ref[...]) + b_ref[...]
    z = jnp.maximum(z, 0.0)
    rows = i * RB + lax.broadcasted_iota(jnp.int32, (RB, 1), 0)
    z = jnp.where(rows < N, z, 0.0)
    h2 = jnp.dot(z, w_ref[...], preferred_element_type=jnp.float32)
    g2_ref[...] = h2 * dinv


def _tc2(a0, a1, g, dinv, b1, w2):
    return pl.pallas_call(
        _tc2_body,
        grid=(GRID,),
        in_specs=[
            pl.BlockSpec((RB, D), lambda i: (i, 0)),
            pl.BlockSpec((RB, D), lambda i: (i, 0)),
            pl.BlockSpec((RB, D), lambda i: (i, 0)),
            pl.BlockSpec((RB, 16), lambda i: (i, 0)),
            pl.BlockSpec((1, D), lambda i: (0, 0)),
            pl.BlockSpec((D, D), lambda i: (0, 0)),
        ],
        out_specs=pl.BlockSpec((RB, D), lambda i: (i, 0)),
        out_shape=jax.ShapeDtypeStruct((NPAD, D), jnp.float32),
    )(a0, a1, g, dinv, b1, w2)


def _tc3_body(a0_ref, a1_ref, g_ref, dinv_ref, b_ref, out_ref):
    dinv = dinv_ref[:, 0:1]
    z = dinv * (a0_ref[...] + a1_ref[...] + g_ref[...]) + b_ref[...]
    out_ref[...] = jnp.maximum(z, 0.0)


def _tc3(a0, a1, g, dinv, b2):
    return pl.pallas_call(
        _tc3_body,
        grid=(GRID,),
        in_specs=[
            pl.BlockSpec((RB, D), lambda i: (i, 0)),
            pl.BlockSpec((RB, D), lambda i: (i, 0)),
            pl.BlockSpec((RB, D), lambda i: (i, 0)),
            pl.BlockSpec((RB, 16), lambda i: (i, 0)),
            pl.BlockSpec((1, D), lambda i: (0, 0)),
        ],
        out_specs=pl.BlockSpec((RB, D), lambda i: (i, 0)),
        out_shape=jax.ShapeDtypeStruct((NPAD, D), jnp.float32),
    )(a0, a1, g, dinv, b2)


def kernel(x, edge_index, W1, b1, W2, b2):
    src = edge_index[0].astype(jnp.int32)
    dst = edge_index[1].astype(jnp.int32)
    pad = NW * EPT_PAD - E
    pad_idx = N + jnp.arange(pad, dtype=jnp.int32) % (NPAD - N)
    src_p = jnp.concatenate([src, pad_idx]).reshape(NW * NSUP, SUP, EC)
    dst_p = jnp.concatenate([dst, pad_idx]).reshape(NW * NSUP, SUP, EC)

    x_p = jnp.pad(x, ((0, NPAD - N), (0, 0)))
    b1r = b1.reshape(1, D)
    b2r = b2.reshape(1, D)

    deg = _deg_call(dst_p)
    d0 = deg[:NPAD]
    d1 = deg[NPAD:]

    g1, dinv = _tc1(x_p, W1, d0, d1)
    acc1 = _scat_call(g1, src_p, dst_p)
    g2 = _tc2(acc1[:NPAD], acc1[NPAD:], g1, dinv, b1r, W2)
    acc2 = _scat_call(g2, src_p, dst_p)
    out = _tc3(acc2[:NPAD], acc2[NPAD:], g2, dinv, b2r)
    return out[:N]
